# Initial kernel scaffold; baseline (speedup 1.0000x reference)
import jax
import jax.numpy as jnp
from jax.experimental import pallas as pl


def kernel(contexts, targets, negative_samples, W_in, W_out):
    raise NotImplementedError("write your pallas kernel here")


# speedup vs baseline: 3.3525x; 3.3525x over previous
import jax
import jax.numpy as jnp
from jax import lax
from jax.experimental import pallas as pl
from jax.experimental.pallas import tpu as pltpu
from jax.experimental.pallas import tpu_sc as plsc

B = 16384
C = 10
K = 20
D = 64
L = 16
NW = 32
RPW = B // NW
NB = 16
NCHUNK = RPW // NB


def _gather_tiles(table_hbm, idx_v, rows_v, n, sem):
    handles = []
    off = 0
    while off < n:
        ln = min(128, n - off)
        handles.append(
            pltpu.async_copy(
                table_hbm.at[idx_v.at[pl.ds(off, ln)]],
                rows_v.at[pl.ds(off, ln)],
                sem,
            )
        )
        off += ln
    return handles


def _body(ctx_pair, ctx_par, tgt_pair, tgt_par, neg_pair, neg_par,
          w_in, w_out, pos_out, neg_out,
          ctx_i_v, ctx_p_v, tgt_i_v, tgt_p_v, neg_i_v, neg_p_v,
          ctx_r_v, tgt_r_v, neg_r_v, mean_v, pos_b_v, neg_b_v, sem):
    wid = lax.axis_index("s") * 2 + lax.axis_index("c")
    lane = lax.broadcasted_iota(jnp.int32, (L,), 0)

    def chunk(ch, carry):
        row0 = wid * RPW + ch * NB
        pltpu.sync_copy(ctx_pair.at[pl.ds(row0 * C, NB * C)], ctx_i_v)
        pltpu.sync_copy(ctx_par.at[pl.ds(row0 * C, NB * C)], ctx_p_v)
        pltpu.sync_copy(tgt_pair.at[pl.ds(row0, NB)], tgt_i_v)
        pltpu.sync_copy(tgt_par.at[pl.ds(row0, NB)], tgt_p_v)
        pltpu.sync_copy(neg_pair.at[pl.ds(row0 * K, NB * K)], neg_i_v)
        pltpu.sync_copy(neg_par.at[pl.ds(row0 * K, NB * K)], neg_p_v)
        handles = (
            _gather_tiles(w_in, ctx_i_v, ctx_r_v, NB * C, sem)
            + _gather_tiles(w_out, tgt_i_v, tgt_r_v, NB, sem)
            + _gather_tiles(w_out, neg_i_v, neg_r_v, NB * K, sem)
        )
        for h in handles:
            h.wait()

        rowsC = lane * C
        rowsK = lane * K

        colb_c = [plsc.load_gather(ctx_p_v, [rowsC + c]) for c in range(C)]

        def dbody1(d, cb):
            s = plsc.load_gather(ctx_r_v, [rowsC + 0, cb[0] + d])
            for c in range(1, C):
                s = s + plsc.load_gather(ctx_r_v, [rowsC + c, cb[c] + d])
            mean_v[pl.ds(d * L, L)] = s * jnp.float32(1.0 / C)
            return cb

        lax.fori_loop(0, D, dbody1, tuple(colb_c))

        colb_t = plsc.load_gather(tgt_p_v, [lane])
        colb_n = [plsc.load_gather(neg_p_v, [rowsK + k]) for k in range(K)]

        def dbody2(d, acc):
            pos, negs, cbt, cbn = acc
            m = mean_v[pl.ds(d * L, L)]
            pos = pos + m * plsc.load_gather(tgt_r_v, [lane, cbt + d])
            negs = tuple(
                negs[k] + m * plsc.load_gather(neg_r_v, [rowsK + k, cbn[k] + d])
                for k in range(K)
            )
            return (pos, negs, cbt, cbn)

        zero = jnp.zeros((L,), jnp.float32)
        pos, negs, _, _ = lax.fori_loop(
            0, D, dbody2, (zero, (zero,) * K, colb_t, tuple(colb_n)))

        pos_b_v[...] = pos
        for k in range(K):
            plsc.store_scatter(neg_b_v, [rowsK + k], negs[k])

        pltpu.sync_copy(pos_b_v, pos_out.at[pl.ds(row0, NB)])
        pltpu.sync_copy(neg_b_v, neg_out.at[pl.ds(row0 * K, NB * K)])
        return carry

    lax.fori_loop(0, NCHUNK, chunk, 0)


@jax.jit
def _cbow_scores(ctx_pair, ctx_par, tgt_pair, tgt_par, neg_pair, neg_par,
                 w_in_p, w_out_p):
    mesh = plsc.VectorSubcoreMesh(core_axis_name="c", subcore_axis_name="s")
    kern = pl.kernel(
        _body,
        out_type=(
            jax.ShapeDtypeStruct((B,), jnp.float32),
            jax.ShapeDtypeStruct((B * K,), jnp.float32),
        ),
        mesh=mesh,
        compiler_params=pltpu.CompilerParams(needs_layout_passes=False),
        scratch_types=[
            pltpu.VMEM((NB * C,), jnp.int32),
            pltpu.VMEM((NB * C,), jnp.int32),
            pltpu.VMEM((NB,), jnp.int32),
            pltpu.VMEM((NB,), jnp.int32),
            pltpu.VMEM((NB * K,), jnp.int32),
            pltpu.VMEM((NB * K,), jnp.int32),
            pltpu.VMEM((NB * C, 128), jnp.float32),
            pltpu.VMEM((NB, 128), jnp.float32),
            pltpu.VMEM((NB * K, 128), jnp.float32),
            pltpu.VMEM((D * L,), jnp.float32),
            pltpu.VMEM((NB,), jnp.float32),
            pltpu.VMEM((NB * K,), jnp.float32),
            pltpu.SemaphoreType.DMA,
        ],
    )
    return kern(ctx_pair, ctx_par, tgt_pair, tgt_par, neg_pair, neg_par,
                w_in_p, w_out_p)


def kernel(contexts, targets, negative_samples, W_in, W_out):
    ctx = contexts.reshape(-1).astype(jnp.int32)
    tgt = targets.reshape(-1).astype(jnp.int32)
    neg = negative_samples.reshape(-1).astype(jnp.int32)
    pos, negf = _cbow_scores(
        ctx >> 1, (ctx & 1) * D,
        tgt >> 1, (tgt & 1) * D,
        neg >> 1, (neg & 1) * D,
        W_in.reshape(-1, 2 * D),
        W_out.reshape(-1, 2 * D),
    )
    return pos, negf.reshape(B, K)


# speedup vs baseline: 3.7173x; 1.1088x over previous
import jax
import jax.numpy as jnp
from jax import lax
from jax.experimental import pallas as pl
from jax.experimental.pallas import tpu as pltpu
from jax.experimental.pallas import tpu_sc as plsc

B = 16384
C = 10
K = 20
D = 64
L = 16
NW = 32
RPW = B // NW
NB = 16
NCHUNK = RPW // NB
NCTX = NB * C
NTN = NB * (1 + K)
IW = 2 * (NCTX + NTN)
O_CTXI = 0
O_CTXP = NCTX
O_TGTI = 2 * NCTX
O_TGTP = O_TGTI + NB
O_NEGI = O_TGTP + NB
O_NEGP = O_NEGI + NB * K
R_TGT = 0
R_NEG = NB


def _ctx_plan():
    return ((O_CTXI + 0, 0, 128), (O_CTXI + 128, 128, 32))


def _tn_plan():
    return (
        (O_TGTI, R_TGT, NB),
        (O_NEGI, R_NEG, 128),
        (O_NEGI + 128, R_NEG + 128, 128),
        (O_NEGI + 256, R_NEG + 256, 64),
    )


def _fire(table_pairs, idx_v, rows_v, sem, plan):
    for io, ro, ln in plan:
        pltpu.async_copy(
            table_pairs.at[idx_v.at[pl.ds(io, ln)]],
            rows_v.at[pl.ds(ro, ln)],
            sem,
        )


def _wait(table_pairs, idx_v, rows_v, sem, plan):
    for io, ro, ln in plan:
        pltpu.make_async_copy(
            table_pairs.at[idx_v.at[pl.ds(io, ln)]],
            rows_v.at[pl.ds(ro, ln)],
            sem,
        ).wait()


def _body(idxcat, w_in, w_out, pos_out, neg_out,
          idx0_v, idx1_v, ctx_r_v, tn0_v, tn1_v, mean_v, pos_b_v, neg_b_v,
          semc, sem0, sem1):
    wid = lax.axis_index("s") * 2 + lax.axis_index("c")
    lane = lax.broadcasted_iota(jnp.int32, (L,), 0)
    bufs = ((idx0_v, tn0_v, sem0), (idx1_v, tn1_v, sem1))

    def stage_idx(ch, idx_v):
        g = wid * NCHUNK + ch
        pltpu.sync_copy(idxcat.at[pl.ds(g * IW, IW)], idx_v)

    def compute(ch, idx_v, tn_v, other_idx):
        row0 = wid * RPW + ch * NB
        rowsC = lane * C
        rowsK = lane * K

        CH = C // 2
        colb_c = tuple(
            plsc.load_gather(idx_v, [O_CTXP + rowsC + c]) for c in range(CH))

        def dbody1a(d, cb):
            s = plsc.load_gather(ctx_r_v, [rowsC, cb[0] + d])
            for c in range(1, CH):
                s = s + plsc.load_gather(ctx_r_v, [rowsC + c, cb[c] + d])
            mean_v[pl.ds(d * L, L)] = s
            return cb

        lax.fori_loop(0, D, dbody1a, colb_c)

        colb_c2 = tuple(
            plsc.load_gather(idx_v, [O_CTXP + rowsC + c]) for c in range(CH, C))

        def dbody1b(d, cb):
            s = mean_v[pl.ds(d * L, L)]
            for c in range(C - CH):
                s = s + plsc.load_gather(ctx_r_v, [rowsC + (CH + c), cb[c] + d])
            mean_v[pl.ds(d * L, L)] = s * jnp.float32(1.0 / C)
            return cb

        lax.fori_loop(0, D, dbody1b, colb_c2)

        @pl.when(ch + 1 < NCHUNK)
        def _():
            _fire(w_in, other_idx, ctx_r_v, semc, _ctx_plan())

        zero = jnp.zeros((L,), jnp.float32)

        colb_t = plsc.load_gather(idx_v, [O_TGTP + lane])

        def dbody2t(d, acc):
            pos, cbt = acc
            m = mean_v[pl.ds(d * L, L)]
            pos = pos + m * plsc.load_gather(tn_v, [R_TGT + lane, cbt + d])
            return (pos, cbt)

        pos, _ = lax.fori_loop(0, D, dbody2t, (zero, colb_t))
        pos_b_v[...] = pos

        KG = 5
        for k0 in range(0, K, KG):
            colb_g = tuple(
                plsc.load_gather(idx_v, [O_NEGP + rowsK + k])
                for k in range(k0, k0 + KG))

            def dbody2n(d, acc, k0=k0):
                negs, cbn = acc
                m = mean_v[pl.ds(d * L, L)]
                negs = tuple(
                    negs[j] + m * plsc.load_gather(
                        tn_v, [R_NEG + rowsK + (k0 + j), cbn[j] + d])
                    for j in range(KG)
                )
                return (negs, cbn)

            negs_g, _ = lax.fori_loop(0, D, dbody2n, ((zero,) * KG, colb_g))
            for j in range(KG):
                plsc.store_scatter(neg_b_v, [rowsK + (k0 + j)], negs_g[j])

        pltpu.sync_copy(pos_b_v, pos_out.at[pl.ds(row0, NB)])
        pltpu.sync_copy(neg_b_v, neg_out.at[pl.ds(row0 * K, NB * K)])

    stage_idx(0, idx0_v)
    _fire(w_in, idx0_v, ctx_r_v, semc, _ctx_plan())
    _fire(w_out, idx0_v, tn0_v, sem0, _tn_plan())
    stage_idx(1, idx1_v)
    _fire(w_out, idx1_v, tn1_v, sem1, _tn_plan())

    def outer(gg, carry):
        for b in (0, 1):
            ch = gg * 2 + b
            idx_v, tn_v, sem = bufs[b]
            _wait(w_in, idx_v, ctx_r_v, semc, _ctx_plan())
            _wait(w_out, idx_v, tn_v, sem, _tn_plan())
            compute(ch, idx_v, tn_v, bufs[1 - b][0])

            @pl.when(ch + 2 < NCHUNK)
            def _():
                stage_idx(ch + 2, idx_v)
                _fire(w_out, idx_v, tn_v, sem, _tn_plan())
        return carry

    lax.fori_loop(0, NCHUNK // 2, outer, 0)


@jax.jit
def _cbow_scores(idxcat, w_in_p, w_out_p):
    mesh = plsc.VectorSubcoreMesh(core_axis_name="c", subcore_axis_name="s")
    kern = pl.kernel(
        _body,
        out_type=(
            jax.ShapeDtypeStruct((B,), jnp.float32),
            jax.ShapeDtypeStruct((B * K,), jnp.float32),
        ),
        mesh=mesh,
        compiler_params=pltpu.CompilerParams(needs_layout_passes=False),
        scratch_types=[
            pltpu.VMEM((IW,), jnp.int32),
            pltpu.VMEM((IW,), jnp.int32),
            pltpu.VMEM((NCTX, 128), jnp.float32),
            pltpu.VMEM((NTN, 128), jnp.float32),
            pltpu.VMEM((NTN, 128), jnp.float32),
            pltpu.VMEM((D * L,), jnp.float32),
            pltpu.VMEM((NB,), jnp.float32),
            pltpu.VMEM((NB * K,), jnp.float32),
            pltpu.SemaphoreType.DMA,
            pltpu.SemaphoreType.DMA,
            pltpu.SemaphoreType.DMA,
        ],
    )
    return kern(idxcat, w_in_p, w_out_p)


def kernel(contexts, targets, negative_samples, W_in, W_out):
    ctx = contexts.reshape(-1).astype(jnp.int32)
    tgt = targets.reshape(-1).astype(jnp.int32)
    neg = negative_samples.reshape(-1).astype(jnp.int32)
    nchunks = B // NB
    idxcat = jnp.concatenate(
        [
            (ctx >> 1).reshape(nchunks, NB * C),
            ((ctx & 1) * D).reshape(nchunks, NB * C),
            (tgt >> 1).reshape(nchunks, NB),
            ((tgt & 1) * D).reshape(nchunks, NB),
            (neg >> 1).reshape(nchunks, NB * K),
            ((neg & 1) * D).reshape(nchunks, NB * K),
        ],
        axis=1,
    ).reshape(-1)
    pos, negf = _cbow_scores(
        idxcat, W_in.reshape(-1, 2 * D), W_out.reshape(-1, 2 * D))
    return pos, negf.reshape(B, K)
